# baseline (speedup 1.0000x reference)
import jax
import jax.numpy as jnp
from jax import lax
from jax.experimental import pallas as pl
from jax.experimental.pallas import tpu as pltpu
from jax.experimental.pallas import tpu_sc as plsc

_B = 16384
_L = 16
_S = 4
_CPS = _B // (_L * _S)
_K = 256
_EPS = 1e-7


def _sc_body(lh_hbm, d_hbm, s_hbm, lhw, dS, key_a, key_b, idx_a, idx_b,
             c0, c1, c2, c3):
    c = lax.axis_index("c")
    s = lax.axis_index("s")
    cnt = [c0, c1, c2, c3]

    @pl.when(jnp.logical_and(c == 0, s == 0))
    def _():
        pltpu.sync_copy(lh_hbm, lhw)
        pltpu.sync_copy(d_hbm, dS)

        lane = lax.iota(jnp.int32, _L)
        base = lane * (_B // _L)
        ones = jnp.ones((_L,), jnp.int32)
        m255 = jnp.full((_L,), 255, jnp.int32)

        def zero_cnt():
            @plsc.parallel_loop(0, _K, unroll=4)
            def _z(g):
                z = jnp.zeros((_L,), jnp.int32)
                for st in range(_S):
                    cnt[st][pl.ds(g * _L, _L)] = z

        @plsc.parallel_loop(0, _B // _L, unroll=4,
                            carry=jnp.full((_L,), -jnp.inf, jnp.float32))
        def mvec(v, m):
            idx_a[pl.ds(v * _L, _L)] = v * _L + lane
            return jnp.maximum(m, lhw[pl.ds(v * _L, _L)])
        gamma = plsc.sort_key_val(mvec, mvec)[0][_L - 1]

        zero_cnt()

        @plsc.parallel_loop(0, _CPS, unroll=2)
        def _f(v):
            for st in range(_S):
                iv = base + (st * _CPS + v)
                d16 = plsc.load_gather(dS, [iv])
                ub = ~plsc.bitcast(d16, jnp.int32)
                plsc.store_scatter(key_a, [iv], ub)
                plsc.addupdate_scatter(cnt[st], [(ub & m255) * _L + lane],
                                       ones)

        def scan_cnt():
            @plsc.parallel_loop(0, _K, unroll=2, carry=jnp.int32(0))
            def _s(g, carry):
                rows = [cnt[st][pl.ds(g * _L, _L)] for st in range(_S)]
                tot = rows[0]
                for st in range(1, _S):
                    tot = tot + rows[st]
                inc = plsc.cumsum(tot)
                e = inc - tot + carry
                for st in range(_S):
                    cnt[st][pl.ds(g * _L, _L)] = e
                    e = e + rows[st]
                return carry + inc[_L - 1]

        def hist(key_src, shift):
            sh = jnp.full((_L,), shift, jnp.int32)
            @plsc.parallel_loop(0, _CPS, unroll=2)
            def _h(v):
                for st in range(_S):
                    iv = base + (st * _CPS + v)
                    k = plsc.load_gather(key_src, [iv])
                    dig = lax.shift_right_logical(k, sh) & m255
                    plsc.addupdate_scatter(cnt[st], [dig * _L + lane], ones)

        def permute(key_src, idx_src, key_dst, idx_dst, shift):
            sh = jnp.full((_L,), shift, jnp.int32)
            def pbody(v, _):
                for st in range(_S):
                    iv = base + (st * _CPS + v)
                    k = plsc.load_gather(key_src, [iv])
                    pay = plsc.load_gather(idx_src, [iv])
                    dig = lax.shift_right_logical(k, sh) & m255
                    slot = dig * _L + lane
                    pos = plsc.load_gather(cnt[st], [slot])
                    plsc.store_scatter(key_dst, [pos], k)
                    plsc.store_scatter(idx_dst, [pos], pay)
                    plsc.addupdate_scatter(cnt[st], [slot], ones)
                return 0
            lax.fori_loop(0, _CPS, pbody, 0)

        scan_cnt()
        permute(key_a, idx_a, key_b, idx_b, 0)

        zero_cnt()
        hist(key_b, 8)
        scan_cnt()
        permute(key_b, idx_b, key_a, idx_a, 8)

        zero_cnt()
        hist(key_a, 16)
        scan_cnt()
        permute(key_a, idx_a, key_b, idx_b, 16)

        zero_cnt()
        hist(key_b, 24)
        scan_cnt()
        permute(key_b, idx_b, key_a, idx_a, 24)

        zf = tuple(jnp.zeros((_L,), jnp.float32) for _ in range(_S))

        @plsc.parallel_loop(0, _B // (_S * _L), unroll=2, carry=zf)
        def tot(v, carry):
            acc = list(carry)
            for st in range(_S):
                sv = idx_a[pl.ds(st * (_B // _S) + v * _L, _L)]
                wv = jnp.exp(plsc.load_gather(lhw, [sv]) - gamma)
                acc[st] = acc[st] + wv
            return tuple(acc)

        offs = []
        run = jnp.float32(0.0)
        for st in range(_S):
            offs.append(run)
            run = run + plsc.cumsum(tot[st])[_L - 1]

        @plsc.parallel_loop(0, _B // (_S * _L), unroll=2, carry=tuple(offs))
        def _f2(v, carry):
            cs = list(carry)
            for st in range(_S):
                sv = idx_a[pl.ds(st * (_B // _S) + v * _L, _L)]
                wv = jnp.exp(plsc.load_gather(lhw, [sv]) - gamma)
                inc = plsc.cumsum(wv) + cs[st]
                plsc.store_scatter(dS, [sv], inc)
                cs[st] = inc[_L - 1]
            return tuple(cs)

        pltpu.sync_copy(dS, s_hbm)


def _risk_set_sums(lh, d):
    mesh = plsc.VectorSubcoreMesh(core_axis_name="c", subcore_axis_name="s")
    return pl.kernel(
        _sc_body,
        out_type=jax.ShapeDtypeStruct((_B,), jnp.float32),
        mesh=mesh,
        compiler_params=pltpu.CompilerParams(needs_layout_passes=False),
        scratch_types=[
            pltpu.VMEM((_B,), jnp.float32),
            pltpu.VMEM((_B,), jnp.float32),
            pltpu.VMEM((_B,), jnp.int32),
            pltpu.VMEM((_B,), jnp.int32),
            pltpu.VMEM((_B,), jnp.int32),
            pltpu.VMEM((_B,), jnp.int32),
            pltpu.VMEM((_K * _L,), jnp.int32),
            pltpu.VMEM((_K * _L,), jnp.int32),
            pltpu.VMEM((_K * _L,), jnp.int32),
            pltpu.VMEM((_K * _L,), jnp.int32),
        ],
    )(lh, d)


def _loss_kernel(lh_ref, e_ref, s_ref, out_ref):
    lh = lh_ref[:, :]
    e = e_ref[:, :]
    srow = s_ref[:, :]
    gamma = jnp.max(lh)
    num = jnp.sum(e * (lh - (jnp.log(srow + _EPS) + gamma)))
    den = jnp.sum(e)
    loss = -num / den
    loss = jnp.where(jnp.isnan(loss), jnp.inf, loss)
    loss = jnp.where(jnp.isneginf(loss), jnp.inf, loss)
    out_ref[0, 0] = loss


def kernel(input, target, weight):
    s = _risk_set_sums(input, target)
    out = pl.pallas_call(
        _loss_kernel,
        out_shape=jax.ShapeDtypeStruct((1, 1), jnp.float32),
        out_specs=pl.BlockSpec(memory_space=pltpu.SMEM),
    )(input.reshape(1, _B), weight.reshape(1, _B), s.reshape(1, _B))
    return out[0, 0]


# speedup vs baseline: 1.2684x; 1.2684x over previous
import jax
import jax.numpy as jnp
from jax import lax
from jax.experimental import pallas as pl
from jax.experimental.pallas import tpu as pltpu
from jax.experimental.pallas import tpu_sc as plsc

_B = 16384
_L = 16
_S = 4
_CPS = _B // (_L * _S)
_K = 256
_EPS = 1e-7


def _sc_body(lh_hbm, d_hbm, s_hbm, lhw, dS, key_a, key_b, idx_a, idx_b, cnt):
    c = lax.axis_index("c")
    s = lax.axis_index("s")

    @pl.when(jnp.logical_and(c == 0, s == 0))
    def _():
        pltpu.sync_copy(lh_hbm, lhw)
        pltpu.sync_copy(d_hbm, dS)

        lane = lax.iota(jnp.int32, _L)
        base = lane * (_B // _L)
        laneS = lane * _S
        ones = jnp.ones((_L,), jnp.int32)
        m255 = jnp.full((_L,), 255, jnp.int32)
        _NSUB = _L * _S

        def zero_cnt():
            @plsc.parallel_loop(0, _K * _S, unroll=4)
            def _z(g):
                cnt[pl.ds(g * _L, _L)] = jnp.zeros((_L,), jnp.int32)

        @plsc.parallel_loop(0, _B // _L, unroll=4,
                            carry=jnp.full((_L,), -jnp.inf, jnp.float32))
        def mvec(v, m):
            idx_a[pl.ds(v * _L, _L)] = v * _L + lane
            return jnp.maximum(m, lhw[pl.ds(v * _L, _L)])
        gamma = plsc.sort_key_val(mvec, mvec)[0][_L - 1]

        zero_cnt()

        @plsc.parallel_loop(0, _CPS, unroll=2)
        def _f(v):
            for st in range(_S):
                iv = base + (st * _CPS + v)
                d16 = plsc.load_gather(dS, [iv])
                ub = ~plsc.bitcast(d16, jnp.int32)
                plsc.store_scatter(key_a, [iv], ub)
                plsc.addupdate_scatter(
                    cnt, [(ub & m255) * _NSUB + laneS + st], ones)

        def scan_cnt():
            @plsc.parallel_loop(0, _K, unroll=2, carry=jnp.int32(0))
            def _s(g, carry):
                ix = [g * _NSUB + laneS + st for st in range(_S)]
                rows = [plsc.load_gather(cnt, [ix[st]]) for st in range(_S)]
                tot = rows[0]
                for st in range(1, _S):
                    tot = tot + rows[st]
                inc = plsc.cumsum(tot)
                e = inc - tot + carry
                for st in range(_S):
                    plsc.store_scatter(cnt, [ix[st]], e)
                    e = e + rows[st]
                return carry + inc[_L - 1]

        def hist(key_src, shift):
            sh = jnp.full((_L,), shift, jnp.int32)
            @plsc.parallel_loop(0, _CPS, unroll=2)
            def _h(v):
                for st in range(_S):
                    iv = base + (st * _CPS + v)
                    k = plsc.load_gather(key_src, [iv])
                    dig = lax.shift_right_logical(k, sh) & m255
                    plsc.addupdate_scatter(
                        cnt, [dig * _NSUB + laneS + st], ones)

        def permute(key_src, idx_src, key_dst, idx_dst, shift):
            sh = jnp.full((_L,), shift, jnp.int32)
            def pbody(v, _):
                @plsc.parallel_loop(0, _S, unroll=_S)
                def _p(st):
                    iv = base + (st * _CPS + v)
                    k = plsc.load_gather(key_src, [iv])
                    pay = plsc.load_gather(idx_src, [iv])
                    dig = lax.shift_right_logical(k, sh) & m255
                    slot = dig * _NSUB + laneS + st
                    pos = plsc.load_gather(cnt, [slot])
                    plsc.store_scatter(key_dst, [pos], k)
                    plsc.store_scatter(idx_dst, [pos], pay)
                    plsc.addupdate_scatter(cnt, [slot], ones)
                return 0
            lax.fori_loop(0, _CPS, pbody, 0)

        scan_cnt()
        permute(key_a, idx_a, key_b, idx_b, 0)

        zero_cnt()
        hist(key_b, 8)
        scan_cnt()
        permute(key_b, idx_b, key_a, idx_a, 8)

        zero_cnt()
        hist(key_a, 16)
        scan_cnt()
        permute(key_a, idx_a, key_b, idx_b, 16)

        zero_cnt()
        hist(key_b, 24)
        scan_cnt()
        permute(key_b, idx_b, key_a, idx_a, 24)

        zf = tuple(jnp.zeros((_L,), jnp.float32) for _ in range(_S))

        @plsc.parallel_loop(0, _B // (_S * _L), unroll=2, carry=zf)
        def tot(v, carry):
            acc = list(carry)
            for st in range(_S):
                sv = idx_a[pl.ds(st * (_B // _S) + v * _L, _L)]
                wv = jnp.exp(plsc.load_gather(lhw, [sv]) - gamma)
                acc[st] = acc[st] + wv
            return tuple(acc)

        offs = []
        run = jnp.float32(0.0)
        for st in range(_S):
            offs.append(run)
            run = run + plsc.cumsum(tot[st])[_L - 1]

        @plsc.parallel_loop(0, _B // (_S * _L), unroll=2, carry=tuple(offs))
        def _f2(v, carry):
            cs = list(carry)
            for st in range(_S):
                sv = idx_a[pl.ds(st * (_B // _S) + v * _L, _L)]
                wv = jnp.exp(plsc.load_gather(lhw, [sv]) - gamma)
                inc = plsc.cumsum(wv) + cs[st]
                plsc.store_scatter(dS, [sv], inc)
                cs[st] = inc[_L - 1]
            return tuple(cs)

        pltpu.sync_copy(dS, s_hbm)


def _risk_set_sums(lh, d):
    mesh = plsc.VectorSubcoreMesh(core_axis_name="c", subcore_axis_name="s")
    return pl.kernel(
        _sc_body,
        out_type=jax.ShapeDtypeStruct((_B,), jnp.float32),
        mesh=mesh,
        compiler_params=pltpu.CompilerParams(needs_layout_passes=False),
        scratch_types=[
            pltpu.VMEM((_B,), jnp.float32),
            pltpu.VMEM((_B,), jnp.float32),
            pltpu.VMEM((_B,), jnp.int32),
            pltpu.VMEM((_B,), jnp.int32),
            pltpu.VMEM((_B,), jnp.int32),
            pltpu.VMEM((_B,), jnp.int32),
            pltpu.VMEM((_K * _L * _S,), jnp.int32),
        ],
    )(lh, d)


def _loss_kernel(lh_ref, e_ref, s_ref, out_ref):
    lh = lh_ref[:, :]
    e = e_ref[:, :]
    srow = s_ref[:, :]
    gamma = jnp.max(lh)
    num = jnp.sum(e * (lh - (jnp.log(srow + _EPS) + gamma)))
    den = jnp.sum(e)
    loss = -num / den
    loss = jnp.where(jnp.isnan(loss), jnp.inf, loss)
    loss = jnp.where(jnp.isneginf(loss), jnp.inf, loss)
    out_ref[0, 0] = loss


def kernel(input, target, weight):
    s = _risk_set_sums(input, target)
    out = pl.pallas_call(
        _loss_kernel,
        out_shape=jax.ShapeDtypeStruct((1, 1), jnp.float32),
        out_specs=pl.BlockSpec(memory_space=pltpu.SMEM),
    )(input.reshape(1, _B), weight.reshape(1, _B), s.reshape(1, _B))
    return out[0, 0]
